# Initial kernel scaffold; baseline (speedup 1.0000x reference)
import jax
import jax.numpy as jnp
from jax.experimental import pallas as pl


def kernel(x, Wg, bg, We, be):
    raise NotImplementedError("write your pallas kernel here")


# speedup vs baseline: 2.9322x; 2.9322x over previous
import functools

import jax
import jax.numpy as jnp
from jax.experimental import pallas as pl


def _moe_body(x_ref, wg_ref, bg_ref, we_ref, be_ref, o_ref, *, bt, e):
    xb = x_ref[...]
    logits = jnp.dot(xb, wg_ref[...], preferred_element_type=jnp.float32)
    logits = logits + bg_ref[...]
    m = jnp.max(logits, axis=-1, keepdims=True)
    p = jnp.exp(logits - m)
    p = p / jnp.sum(p, axis=-1, keepdims=True)

    iota = jax.lax.broadcasted_iota(jnp.int32, (bt, e), 1)
    v0 = jnp.max(p, axis=-1, keepdims=True)
    i0 = jnp.min(jnp.where(p >= v0, iota, e), axis=-1, keepdims=True)
    sel0 = iota == i0
    p2 = jnp.where(sel0, -jnp.inf, p)
    v1 = jnp.max(p2, axis=-1, keepdims=True)
    i1 = jnp.min(jnp.where(p2 >= v1, iota, e), axis=-1, keepdims=True)
    sel1 = iota == i1
    g = jnp.where(sel0, v0, 0.0) + jnp.where(sel1, v1, 0.0)

    acc = jnp.dot(g, be_ref[...], preferred_element_type=jnp.float32)
    for ei in range(e):
        acc = acc + jnp.dot(g[:, ei:ei + 1] * xb, we_ref[ei],
                            preferred_element_type=jnp.float32)
    o_ref[...] = acc


def kernel(x, Wg, bg, We, be):
    T, D = x.shape
    E = Wg.shape[1]
    BT = 512
    body = functools.partial(_moe_body, bt=BT, e=E)
    return pl.pallas_call(
        body,
        grid=(T // BT,),
        in_specs=[
            pl.BlockSpec((BT, D), lambda i: (i, 0)),
            pl.BlockSpec((D, E), lambda i: (0, 0)),
            pl.BlockSpec((1, E), lambda i: (0, 0)),
            pl.BlockSpec((E, D, D), lambda i: (0, 0, 0)),
            pl.BlockSpec((E, D), lambda i: (0, 0)),
        ],
        out_specs=pl.BlockSpec((BT, D), lambda i: (i, 0)),
        out_shape=jax.ShapeDtypeStruct((T, D), jnp.float32),
    )(x, Wg, bg.reshape(1, E), We, be)
